# baseline (speedup 1.0000x reference)
import functools

import jax
import jax.numpy as jnp
from jax import lax
from jax.experimental import pallas as pl
from jax.experimental.pallas import tpu as pltpu
from jax.experimental.pallas import tpu_sc as plsc

T, B = 512, 16
D, H, OUT = 512, 512, 128

CT = 128
NCHUNK = T // CT

SC_CORES = 2
SC_SUBCORES = 16
NW = SC_CORES * SC_SUBCORES
ROWS_PER_W = (T * B) // NW
GCH = 64


def _sc_gather_body(table_hbm, idx_hbm, out_hbm, idx_v, rows_v, sem):
    wid = lax.axis_index("s") * SC_CORES + lax.axis_index("c")
    base = wid * ROWS_PER_W
    for c in range(ROWS_PER_W // GCH):
        off = base + c * GCH
        pltpu.sync_copy(idx_hbm.at[pl.ds(off, GCH)], idx_v)
        pltpu.async_copy(table_hbm.at[idx_v], rows_v, sem).wait()
        pltpu.sync_copy(rows_v, out_hbm.at[pl.ds(off, GCH)])


def _sc_gather(table, idx):
    mesh = plsc.VectorSubcoreMesh(core_axis_name="c", subcore_axis_name="s")
    gk = functools.partial(
        pl.kernel,
        mesh=mesh,
        out_type=jax.ShapeDtypeStruct((T * B, D), jnp.float32),
        scratch_types=[
            pltpu.VMEM((GCH,), jnp.int32),
            pltpu.VMEM((GCH, D), jnp.float32),
            pltpu.SemaphoreType.DMA,
        ],
    )(_sc_gather_body)
    return gk(table, idx)


def _rnn_body(x0_ref, xb_ref, wih_ref, whh_ref, bias_ref, len_ref, h2o_ref,
              h2ob_ref, out_ref, xp_ref, h_ref, max_ref):
    i = pl.program_id(0)
    cur = lax.rem(i, 2)
    nxt = 1 - cur

    @pl.when(i == 0)
    def _init():
        h_ref[...] = jnp.zeros_like(h_ref)
        max_ref[...] = jnp.full_like(max_ref, -jnp.inf)
        xp_ref[0] = (
            jnp.dot(x0_ref[...], wih_ref[...],
                    preferred_element_type=jnp.float32)
            + bias_ref[...]
        )

    def step(t, carry):
        h, mx = carry
        hw = jnp.dot(h, whh_ref[...], preferred_element_type=jnp.float32)
        hn = jnp.tanh(xp_ref[cur, pl.ds(t * B, B), :] + hw)
        mask = (i * CT + t) < len_ref[...]
        mx = jnp.where(mask, jnp.maximum(mx, hn), mx)
        xp_ref[nxt, pl.ds(t * B, B), :] = (
            jnp.dot(xb_ref[pl.ds(t * B, B), :], wih_ref[...],
                    preferred_element_type=jnp.float32)
            + bias_ref[...]
        )
        return (hn, mx)

    hf, mxf = lax.fori_loop(0, CT, step, (h_ref[...], max_ref[...]), unroll=32)
    h_ref[...] = hf
    max_ref[...] = mxf

    @pl.when(i == NCHUNK - 1)
    def _fin():
        out_ref[...] = (
            jnp.dot(max_ref[...], h2o_ref[...], preferred_element_type=jnp.float32)
            + h2ob_ref[...]
        )


def _rnn_call(x, wihT, whhT, bias, lenb, h2oT, h2ob):
    return pl.pallas_call(
        _rnn_body,
        grid=(NCHUNK,),
        in_specs=[
            pl.BlockSpec((CT * B, D), lambda i: (0, 0)),
            pl.BlockSpec((CT * B, D),
                         lambda i: (jnp.minimum(i + 1, NCHUNK - 1), 0)),
            pl.BlockSpec((D, H), lambda i: (0, 0)),
            pl.BlockSpec((H, H), lambda i: (0, 0)),
            pl.BlockSpec((1, H), lambda i: (0, 0)),
            pl.BlockSpec((B, H), lambda i: (0, 0)),
            pl.BlockSpec((H, OUT), lambda i: (0, 0)),
            pl.BlockSpec((1, OUT), lambda i: (0, 0)),
        ],
        out_specs=pl.BlockSpec((B, OUT), lambda i: (0, 0)),
        out_shape=jax.ShapeDtypeStruct((B, OUT), jnp.float32),
        scratch_shapes=[
            pltpu.VMEM((2, CT * B, H), jnp.float32),
            pltpu.VMEM((B, H), jnp.float32),
            pltpu.VMEM((B, H), jnp.float32),
        ],
    )(x, x, wihT, whhT, bias, lenb, h2oT, h2ob)


def kernel(input_, input_lengths, embed_table, W_ih, W_hh, b_ih, b_hh, h2o_w, h2o_b):
    idx = input_.reshape(T * B).astype(jnp.int32)
    gathered = _sc_gather(embed_table, idx)
    bias = (b_ih + b_hh).reshape(1, H)
    lenb = jnp.broadcast_to(
        input_lengths.astype(jnp.int32).reshape(B, 1), (B, H)
    )
    return _rnn_call(
        gathered, W_ih.T, W_hh.T, bias, lenb, h2o_w.T, h2o_b.reshape(1, OUT)
    )


# speedup vs baseline: 1.0483x; 1.0483x over previous
import functools

import jax
import jax.numpy as jnp
from jax import lax
from jax.experimental import pallas as pl
from jax.experimental.pallas import tpu as pltpu
from jax.experimental.pallas import tpu_sc as plsc

T, B = 512, 16
D, H, OUT = 512, 512, 128

CT = 64
NCHUNK = T // CT

SC_CORES = 2
SC_SUBCORES = 16
NW = SC_CORES * SC_SUBCORES
ROWS_PER_W = (T * B) // NW
GCH = 64


def _sc_gather_body(table_hbm, idx_hbm, out_hbm, idx0, idx1, rows0, rows1,
                    sem0, sem1):
    wid = lax.axis_index("s") * SC_CORES + lax.axis_index("c")
    base = wid * ROWS_PER_W
    idxb, rowsb, sems = (idx0, idx1), (rows0, rows1), (sem0, sem1)
    nch = ROWS_PER_W // GCH
    pltpu.sync_copy(idx_hbm.at[pl.ds(base, GCH)], idx0)
    cps = [pltpu.async_copy(table_hbm.at[idx0], rows0, sem0), None]
    for c in range(nch):
        p, q = c % 2, (c + 1) % 2
        if c + 1 < nch:
            off1 = base + (c + 1) * GCH
            pltpu.sync_copy(idx_hbm.at[pl.ds(off1, GCH)], idxb[q])
            cps[q] = pltpu.async_copy(table_hbm.at[idxb[q]], rowsb[q], sems[q])
        cps[p].wait()
        pltpu.sync_copy(rowsb[p], out_hbm.at[pl.ds(base + c * GCH, GCH)])


def _sc_gather(table, idx):
    mesh = plsc.VectorSubcoreMesh(core_axis_name="c", subcore_axis_name="s")
    gk = functools.partial(
        pl.kernel,
        mesh=mesh,
        out_type=jax.ShapeDtypeStruct((T * B, D), jnp.float32),
        scratch_types=[
            pltpu.VMEM((GCH,), jnp.int32),
            pltpu.VMEM((GCH,), jnp.int32),
            pltpu.VMEM((GCH, D), jnp.float32),
            pltpu.VMEM((GCH, D), jnp.float32),
            pltpu.SemaphoreType.DMA,
            pltpu.SemaphoreType.DMA,
        ],
    )(_sc_gather_body)
    return gk(table, idx)


def _rnn_body(x0_ref, xb_ref, wih_ref, whh_ref, bias_ref, len_ref, h2o_ref,
              h2ob_ref, out_ref, xp_ref, h_ref, max_ref):
    i = pl.program_id(0)
    cur = lax.rem(i, 2)
    nxt = 1 - cur

    @pl.when(i == 0)
    def _init():
        h_ref[...] = jnp.zeros_like(h_ref)
        max_ref[...] = jnp.full_like(max_ref, -jnp.inf)
        xp_ref[0] = (
            jnp.dot(x0_ref[...], wih_ref[...],
                    preferred_element_type=jnp.float32)
            + bias_ref[...]
        )

    def step(t, carry):
        h, mx = carry
        hw = jnp.dot(h, whh_ref[...], preferred_element_type=jnp.float32)
        hn = jnp.tanh(xp_ref[cur, pl.ds(t * B, B), :] + hw)
        mask = (i * CT + t) < len_ref[...]
        mx = jnp.where(mask, jnp.maximum(mx, hn), mx)
        xp_ref[nxt, pl.ds(t * B, B), :] = (
            jnp.dot(xb_ref[pl.ds(t * B, B), :], wih_ref[...],
                    preferred_element_type=jnp.float32)
            + bias_ref[...]
        )
        return (hn, mx)

    hf, mxf = lax.fori_loop(0, CT, step, (h_ref[...], max_ref[...]), unroll=64)
    h_ref[...] = hf
    max_ref[...] = mxf

    @pl.when(i == NCHUNK - 1)
    def _fin():
        out_ref[...] = (
            jnp.dot(max_ref[...], h2o_ref[...], preferred_element_type=jnp.float32)
            + h2ob_ref[...]
        )


def _rnn_call(x, wihT, whhT, bias, lenb, h2oT, h2ob):
    return pl.pallas_call(
        _rnn_body,
        grid=(NCHUNK,),
        in_specs=[
            pl.BlockSpec((CT * B, D), lambda i: (0, 0)),
            pl.BlockSpec((CT * B, D),
                         lambda i: (jnp.minimum(i + 1, NCHUNK - 1), 0)),
            pl.BlockSpec((D, H), lambda i: (0, 0)),
            pl.BlockSpec((H, H), lambda i: (0, 0)),
            pl.BlockSpec((1, H), lambda i: (0, 0)),
            pl.BlockSpec((B, H), lambda i: (0, 0)),
            pl.BlockSpec((H, OUT), lambda i: (0, 0)),
            pl.BlockSpec((1, OUT), lambda i: (0, 0)),
        ],
        out_specs=pl.BlockSpec((B, OUT), lambda i: (0, 0)),
        out_shape=jax.ShapeDtypeStruct((B, OUT), jnp.float32),
        scratch_shapes=[
            pltpu.VMEM((2, CT * B, H), jnp.float32),
            pltpu.VMEM((B, H), jnp.float32),
            pltpu.VMEM((B, H), jnp.float32),
        ],
    )(x, x, wihT, whhT, bias, lenb, h2oT, h2ob)


def kernel(input_, input_lengths, embed_table, W_ih, W_hh, b_ih, b_hh, h2o_w, h2o_b):
    idx = input_.reshape(T * B).astype(jnp.int32)
    gathered = _sc_gather(embed_table, idx)
    bias = (b_ih + b_hh).reshape(1, H)
    lenb = jnp.broadcast_to(
        input_lengths.astype(jnp.int32).reshape(B, 1), (B, H)
    )
    return _rnn_call(
        gathered, W_ih.T, W_hh.T, bias, lenb, h2o_w.T, h2o_b.reshape(1, OUT)
    )


# speedup vs baseline: 1.0644x; 1.0154x over previous
import functools

import jax
import jax.numpy as jnp
from jax import lax
from jax.experimental import pallas as pl
from jax.experimental.pallas import tpu as pltpu
from jax.experimental.pallas import tpu_sc as plsc

T, B = 512, 16
D, H, OUT = 512, 512, 128

CT = 64
HALF = T // 2
NG = HALF // CT

SC_CORES = 2
SC_SUBCORES = 16
NW = SC_CORES * SC_SUBCORES
GCH = 64


def _sc_gather(table, idx):
    nrows = idx.shape[0]
    rpw = nrows // NW
    nch = rpw // GCH

    def body(table_hbm, idx_hbm, out_hbm, idx0, idx1, rows0, rows1,
             sem0, sem1):
        wid = lax.axis_index("s") * SC_CORES + lax.axis_index("c")
        base = wid * rpw
        idxb, rowsb, sems = (idx0, idx1), (rows0, rows1), (sem0, sem1)
        pltpu.sync_copy(idx_hbm.at[pl.ds(base, GCH)], idx0)
        cps = [pltpu.async_copy(table_hbm.at[idx0], rows0, sem0), None]
        for c in range(nch):
            p, q = c % 2, (c + 1) % 2
            if c + 1 < nch:
                off1 = base + (c + 1) * GCH
                pltpu.sync_copy(idx_hbm.at[pl.ds(off1, GCH)], idxb[q])
                cps[q] = pltpu.async_copy(table_hbm.at[idxb[q]], rowsb[q],
                                          sems[q])
            cps[p].wait()
            pltpu.sync_copy(rowsb[p], out_hbm.at[pl.ds(base + c * GCH, GCH)])

    mesh = plsc.VectorSubcoreMesh(core_axis_name="c", subcore_axis_name="s")
    gk = functools.partial(
        pl.kernel,
        mesh=mesh,
        out_type=jax.ShapeDtypeStruct((nrows, D), jnp.float32),
        scratch_types=[
            pltpu.VMEM((GCH,), jnp.int32),
            pltpu.VMEM((GCH,), jnp.int32),
            pltpu.VMEM((GCH, D), jnp.float32),
            pltpu.VMEM((GCH, D), jnp.float32),
            pltpu.SemaphoreType.DMA,
            pltpu.SemaphoreType.DMA,
        ],
    )(body)
    return gk(table, idx)


def _recur(i, toff, x0_ref, xb_ref, wih_ref, whh_ref, bias_ref, len_ref,
           xp_ref, h_ref, max_ref, hin_ref=None, min_ref=None):
    cur = lax.rem(i, 2)
    nxt = 1 - cur

    @pl.when(i == 0)
    def _init():
        if hin_ref is None:
            h_ref[...] = jnp.zeros_like(h_ref)
            max_ref[...] = jnp.full_like(max_ref, -jnp.inf)
        else:
            h_ref[...] = hin_ref[...]
            max_ref[...] = min_ref[...]
        xp_ref[0] = (
            jnp.dot(x0_ref[...], wih_ref[...],
                    preferred_element_type=jnp.float32)
            + bias_ref[...]
        )

    def step(t, carry):
        h, mx = carry
        hw = jnp.dot(h, whh_ref[...], preferred_element_type=jnp.float32)
        hn = jnp.tanh(xp_ref[cur, pl.ds(t * B, B), :] + hw)
        mask = (toff + i * CT + t) < len_ref[...]
        mx = jnp.where(mask, jnp.maximum(mx, hn), mx)
        xp_ref[nxt, pl.ds(t * B, B), :] = (
            jnp.dot(xb_ref[pl.ds(t * B, B), :], wih_ref[...],
                    preferred_element_type=jnp.float32)
            + bias_ref[...]
        )
        return (hn, mx)

    hf, mxf = lax.fori_loop(0, CT, step, (h_ref[...], max_ref[...]), unroll=64)
    h_ref[...] = hf
    max_ref[...] = mxf


def _rnn_body_a(x0_ref, xb_ref, wih_ref, whh_ref, bias_ref, len_ref,
                hout_ref, mout_ref, xp_ref, h_ref, max_ref):
    i = pl.program_id(0)
    _recur(i, 0, x0_ref, xb_ref, wih_ref, whh_ref, bias_ref, len_ref,
           xp_ref, h_ref, max_ref)

    @pl.when(i == NG - 1)
    def _fin():
        hout_ref[...] = h_ref[...]
        mout_ref[...] = max_ref[...]


def _rnn_body_b(x0_ref, xb_ref, wih_ref, whh_ref, bias_ref, len_ref,
                hin_ref, min_ref, h2o_ref, h2ob_ref, out_ref,
                xp_ref, h_ref, max_ref):
    i = pl.program_id(0)
    _recur(i, HALF, x0_ref, xb_ref, wih_ref, whh_ref, bias_ref, len_ref,
           xp_ref, h_ref, max_ref, hin_ref=hin_ref, min_ref=min_ref)

    @pl.when(i == NG - 1)
    def _fin():
        out_ref[...] = (
            jnp.dot(max_ref[...], h2o_ref[...],
                    preferred_element_type=jnp.float32)
            + h2ob_ref[...]
        )


_COMMON_SPECS = [
    pl.BlockSpec((CT * B, D), lambda i: (0, 0)),
    pl.BlockSpec((CT * B, D), lambda i: (jnp.minimum(i + 1, NG - 1), 0)),
    pl.BlockSpec((D, H), lambda i: (0, 0)),
    pl.BlockSpec((H, H), lambda i: (0, 0)),
    pl.BlockSpec((1, H), lambda i: (0, 0)),
    pl.BlockSpec((B, H), lambda i: (0, 0)),
]

_SCRATCH = [
    pltpu.VMEM((2, CT * B, H), jnp.float32),
    pltpu.VMEM((B, H), jnp.float32),
    pltpu.VMEM((B, H), jnp.float32),
]


def _rnn_call_a(x, wihT, whhT, bias, lenb):
    return pl.pallas_call(
        _rnn_body_a,
        grid=(NG,),
        in_specs=list(_COMMON_SPECS),
        out_specs=(
            pl.BlockSpec((B, H), lambda i: (0, 0)),
            pl.BlockSpec((B, H), lambda i: (0, 0)),
        ),
        out_shape=(
            jax.ShapeDtypeStruct((B, H), jnp.float32),
            jax.ShapeDtypeStruct((B, H), jnp.float32),
        ),
        scratch_shapes=list(_SCRATCH),
    )(x, x, wihT, whhT, bias, lenb)


def _rnn_call_b(x, wihT, whhT, bias, lenb, hin, mxin, h2oT, h2ob):
    return pl.pallas_call(
        _rnn_body_b,
        grid=(NG,),
        in_specs=list(_COMMON_SPECS) + [
            pl.BlockSpec((B, H), lambda i: (0, 0)),
            pl.BlockSpec((B, H), lambda i: (0, 0)),
            pl.BlockSpec((H, OUT), lambda i: (0, 0)),
            pl.BlockSpec((1, OUT), lambda i: (0, 0)),
        ],
        out_specs=pl.BlockSpec((B, OUT), lambda i: (0, 0)),
        out_shape=jax.ShapeDtypeStruct((B, OUT), jnp.float32),
        scratch_shapes=list(_SCRATCH),
    )(x, x, wihT, whhT, bias, lenb, hin, mxin, h2oT, h2ob)


def kernel(input_, input_lengths, embed_table, W_ih, W_hh, b_ih, b_hh, h2o_w, h2o_b):
    idx = input_.reshape(T * B).astype(jnp.int32)
    ga = _sc_gather(embed_table, idx[: HALF * B])
    gb = _sc_gather(embed_table, idx[HALF * B:])
    wihT, whhT, h2oT = W_ih.T, W_hh.T, h2o_w.T
    bias = (b_ih + b_hh).reshape(1, H)
    lenb = jnp.broadcast_to(
        input_lengths.astype(jnp.int32).reshape(B, 1), (B, H)
    )
    h1, mx1 = _rnn_call_a(ga, wihT, whhT, bias, lenb)
    return _rnn_call_b(gb, wihT, whhT, bias, lenb, h1, mx1, h2oT,
                       h2o_b.reshape(1, OUT))
